# Initial kernel scaffold; baseline (speedup 1.0000x reference)
import jax
import jax.numpy as jnp
from jax.experimental import pallas as pl


def kernel(u, in_proj_wT, conv_w_klc, conv_b, A_log, D, dt_bias, norm_w, out_proj_wT):
    raise NotImplementedError("write your pallas kernel here")


# speedup vs baseline: 3.3652x; 3.3652x over previous
import functools

import jax
import jax.numpy as jnp
from jax import lax
from jax.experimental import pallas as pl
from jax.experimental.pallas import tpu as pltpu


def _sigmoid(x):
    return 1.0 / (1.0 + jnp.exp(-x))


def _softplus(x):
    return jnp.maximum(x, 0.0) + jnp.log(1.0 + jnp.exp(-jnp.abs(x)))


def _inproj_kernel(x_ref, w_ref, o_ref):
    o_ref[...] = jnp.dot(x_ref[...], w_ref[...],
                         preferred_element_type=jnp.float32)


def _inproj(x_bf16, w_bf16, *, tm=2048, tn=512):
    M, K = x_bf16.shape
    _, N = w_bf16.shape
    grid_m = (M + tm - 1) // tm
    grid_n = (N + tn - 1) // tn
    return pl.pallas_call(
        _inproj_kernel,
        out_shape=jax.ShapeDtypeStruct((M, N), jnp.float32),
        grid=(grid_m, grid_n),
        in_specs=[
            pl.BlockSpec((tm, K), lambda i, j: (i, 0)),
            pl.BlockSpec((K, tn), lambda i, j: (0, j)),
        ],
        out_specs=pl.BlockSpec((tm, tn), lambda i, j: (i, j)),
        compiler_params=pltpu.CompilerParams(
            dimension_semantics=("parallel", "parallel"),
            vmem_limit_bytes=50 * 1024 * 1024),
    )(x_bf16, w_bf16)


def _conv_kernel(x_ref, w_ref, b_ref, o_ref):
    L, C = o_ref.shape[1], o_ref.shape[2]
    K = w_ref.shape[0]
    x = x_ref[0]
    w = w_ref[:, 0, :]
    acc = x * w[K - 1:K, :] + b_ref[...]
    for d in range(1, K):
        sh = jnp.concatenate(
            [jnp.zeros((d, C), jnp.float32), x[: L - d, :]], axis=0)
        acc = acc + sh * w[K - 1 - d:K - d, :]
    o_ref[0] = (acc * _sigmoid(acc)).astype(o_ref.dtype)


def _conv_silu(zxbcdt_3d, w_klc, conv_b, *, col0, conv_dim, cc=256):
    Bsz, L, _ = zxbcdt_3d.shape
    K = w_klc.shape[0]
    assert col0 % cc == 0 and conv_dim % cc == 0
    t0 = col0 // cc
    return pl.pallas_call(
        _conv_kernel,
        out_shape=jax.ShapeDtypeStruct((Bsz, L, conv_dim), jnp.bfloat16),
        grid=(Bsz, conv_dim // cc),
        in_specs=[
            pl.BlockSpec((1, L, cc), lambda b, c: (b, 0, t0 + c)),
            pl.BlockSpec((K, 1, cc), lambda b, c: (0, 0, c)),
            pl.BlockSpec((1, cc), lambda b, c: (0, c)),
        ],
        out_specs=pl.BlockSpec((1, L, cc), lambda b, c: (b, 0, c)),
        compiler_params=pltpu.CompilerParams(
            dimension_semantics=("parallel", "parallel")),
    )(zxbcdt_3d, w_klc, conv_b.reshape(1, conv_dim))


def _ssd_kernel(A_ref, D_ref, dtb_ref, dt_ref, xbc_ref, Bm_ref, Cm_ref,
                y_ref, state_ref, xw_ref, *, headdim):
    P = headdim
    HT = A_ref.shape[-1]
    Q = xbc_ref.shape[1]

    @pl.when(pl.program_id(2) == 0)
    def _init():
        state_ref[...] = jnp.zeros_like(state_ref)

    A = A_ref[0]
    Dv = D_ref[0]
    dtb = dtb_ref[0]
    dt_raw = dt_ref[0, 0]
    x = xbc_ref[0]
    Bg = Bm_ref[0]
    Cg = Cm_ref[0]

    dt = _softplus(dt_raw + dtb)
    a = dt * A

    idx_i = lax.broadcasted_iota(jnp.int32, (Q, Q), 0)
    idx_j = lax.broadcasted_iota(jnp.int32, (Q, Q), 1)
    causal = idx_i >= idx_j
    tri = causal.astype(jnp.float32)

    cA = jnp.dot(tri, a, preferred_element_type=jnp.float32)
    cAT = cA.T
    exp_cA = jnp.exp(cA)
    cA_last = cA[Q - 1:Q, :]
    exp_last = jnp.exp(cA_last)
    w_all = jnp.exp(cA_last - cA) * dt

    BgT = Bg.T
    scores = jnp.dot(Cg, BgT, preferred_element_type=jnp.float32)
    y_inter = jnp.dot(Cg, state_ref[...].astype(jnp.bfloat16),
                      preferred_element_type=jnp.float32)

    neg_big = jnp.float32(-1e30)
    for h in range(HT):
        sl = slice(h * P, (h + 1) * P)
        xh = x[:, sl]
        xf = xh.astype(jnp.float32)
        xdt = xf * dt[:, h:h + 1]

        decay = jnp.exp(jnp.where(causal,
                                  cA[:, h:h + 1] - cAT[h:h + 1, :], neg_big))
        y_h = jnp.dot((scores * decay).astype(jnp.bfloat16),
                      xdt.astype(jnp.bfloat16),
                      preferred_element_type=jnp.float32)

        y_h = y_h + exp_cA[:, h:h + 1] * y_inter[:, sl]
        y_h = y_h + Dv[:, h:h + 1] * xf
        y_ref[0, :, sl] = y_h.astype(y_ref.dtype)

        xw_ref[:, sl] = (xf * w_all[:, h:h + 1]).astype(jnp.bfloat16)

    dS = jnp.dot(BgT, xw_ref[...], preferred_element_type=jnp.float32)
    for h in range(HT):
        sl = slice(h * P, (h + 1) * P)
        state_ref[:, sl] = exp_last[:, h:h + 1] * state_ref[:, sl] + dS[:, sl]


def _ssd_scan(xbc, dt_t, A, Dv, dtb, *, nheads, headdim, d_state, chunk):
    Bsz, L, _ = xbc.shape
    H, P, N, Q = nheads, headdim, d_state, chunk
    d_inner = H * P
    HT = dt_t.shape[-1]
    n_tiles = H // HT
    nC = L // Q
    bcol = d_inner // (HT * P)
    assert d_inner % (HT * P) == 0 and L % Q == 0

    kfn = functools.partial(_ssd_kernel, headdim=P)
    return pl.pallas_call(
        kfn,
        out_shape=jax.ShapeDtypeStruct((Bsz, L, d_inner), jnp.float32),
        grid=(Bsz, n_tiles, nC),
        in_specs=[
            pl.BlockSpec((1, 1, HT), lambda b, t, c: (t, 0, 0)),
            pl.BlockSpec((1, 1, HT), lambda b, t, c: (t, 0, 0)),
            pl.BlockSpec((1, 1, HT), lambda b, t, c: (t, 0, 0)),
            pl.BlockSpec((1, 1, Q, HT), lambda b, t, c: (b, t, c, 0)),
            pl.BlockSpec((1, Q, HT * P), lambda b, t, c: (b, c, t)),
            pl.BlockSpec((1, Q, N),
                         lambda b, t, c: (b, c, bcol * (HT * P) // N)),
            pl.BlockSpec((1, Q, N),
                         lambda b, t, c: (b, c, bcol * (HT * P) // N + 1)),
        ],
        out_specs=pl.BlockSpec((1, Q, HT * P), lambda b, t, c: (b, c, t)),
        scratch_shapes=[pltpu.VMEM((N, HT * P), jnp.float32),
                        pltpu.VMEM((Q, HT * P), jnp.bfloat16)],
        compiler_params=pltpu.CompilerParams(
            dimension_semantics=("parallel", "parallel", "arbitrary"),
            vmem_limit_bytes=24 * 1024 * 1024),
    )(A, Dv, dtb, dt_t, xbc, xbc, xbc)


def _norm_proj_kernel(y_ref, z_ref, nw_ref, w_ref, o_ref):
    y = y_ref[...]
    z = z_ref[...]
    x = y * (z * _sigmoid(z))
    var = jnp.mean(x * x, axis=-1, keepdims=True)
    xn = x * lax.rsqrt(var + 1e-5) * nw_ref[...]
    o_ref[...] = jnp.dot(xn.astype(jnp.bfloat16), w_ref[...],
                         preferred_element_type=jnp.float32)


def _norm_proj(y2d, z_src, norm_w, w_bf16, *, tm=128):
    M, D = y2d.shape
    _, N = w_bf16.shape
    return pl.pallas_call(
        _norm_proj_kernel,
        out_shape=jax.ShapeDtypeStruct((M, N), jnp.float32),
        grid=(M // tm,),
        in_specs=[
            pl.BlockSpec((tm, D), lambda i: (i, 0)),
            pl.BlockSpec((tm, D), lambda i: (i, 0)),
            pl.BlockSpec((1, D), lambda i: (0, 0)),
            pl.BlockSpec((D, N), lambda i: (0, 0)),
        ],
        out_specs=pl.BlockSpec((tm, N), lambda i: (i, 0)),
        compiler_params=pltpu.CompilerParams(
            dimension_semantics=("parallel",),
            vmem_limit_bytes=56 * 1024 * 1024),
    )(y2d, z_src, norm_w.reshape(1, D), w_bf16)


def kernel(u, in_proj_wT, conv_w_klc, conv_b, A_log, D, dt_bias, norm_w,
           out_proj_wT):
    d_model, d_inner, d_state = 2048, 4096, 128
    H, P, G, K = 64, 64, 1, 4
    HT = 16
    chunk = 128
    n_tiles = H // HT
    conv_dim = d_inner + 2 * G * d_state
    d_in_proj = 2 * d_inner + 2 * G * d_state + H

    Bsz, L, _ = u.shape
    M = Bsz * L

    zxbcdt = _inproj(u.reshape(M, d_model).astype(jnp.bfloat16), in_proj_wT)

    xbc = _conv_silu(zxbcdt.reshape(Bsz, L, d_in_proj), conv_w_klc, conv_b,
                     col0=d_inner, conv_dim=conv_dim)

    dt_raw = zxbcdt[:, d_inner + conv_dim:]
    dt_t = dt_raw.reshape(Bsz, L, n_tiles, HT).transpose(0, 2, 1, 3)

    A = (-jnp.exp(A_log)).reshape(n_tiles, 1, HT).astype(jnp.float32)
    Dv = D.reshape(n_tiles, 1, HT).astype(jnp.float32)
    dtb = dt_bias.reshape(n_tiles, 1, HT).astype(jnp.float32)

    y = _ssd_scan(xbc, dt_t, A, Dv, dtb, nheads=H, headdim=P,
                  d_state=d_state, chunk=chunk)

    out = _norm_proj(y.reshape(M, d_inner), zxbcdt, norm_w, out_proj_wT)
    return out.reshape(Bsz, L, d_model)


# speedup vs baseline: 5.0944x; 1.5139x over previous
import functools

import jax
import jax.numpy as jnp
from jax import lax
from jax.experimental import pallas as pl
from jax.experimental.pallas import tpu as pltpu


def _sigmoid(x):
    return 1.0 / (1.0 + jnp.exp(-x))


def _softplus(x):
    return jnp.maximum(x, 0.0) + jnp.log(1.0 + jnp.exp(-jnp.abs(x)))


def _inproj_kernel(x_ref, w_ref, o_ref):
    o_ref[...] = jnp.dot(x_ref[...], w_ref[...],
                         preferred_element_type=jnp.float32)


def _inproj(x_bf16, w_bf16, *, tm=2048, tn=512):
    M, K = x_bf16.shape
    _, N = w_bf16.shape
    grid_m = (M + tm - 1) // tm
    grid_n = (N + tn - 1) // tn
    return pl.pallas_call(
        _inproj_kernel,
        out_shape=jax.ShapeDtypeStruct((M, N), jnp.float32),
        grid=(grid_m, grid_n),
        in_specs=[
            pl.BlockSpec((tm, K), lambda i, j: (i, 0)),
            pl.BlockSpec((K, tn), lambda i, j: (0, j)),
        ],
        out_specs=pl.BlockSpec((tm, tn), lambda i, j: (i, j)),
        compiler_params=pltpu.CompilerParams(
            dimension_semantics=("parallel", "arbitrary"),
            vmem_limit_bytes=50 * 1024 * 1024),
    )(x_bf16, w_bf16)


def _conv_kernel(x_ref, w_ref, b_ref, o_ref):
    L, C = o_ref.shape[1], o_ref.shape[2]
    K = w_ref.shape[0]
    x = x_ref[0]
    w = w_ref[:, 0, :]
    acc = x * w[K - 1:K, :] + b_ref[...]
    for d in range(1, K):
        sh = jnp.concatenate(
            [jnp.zeros((d, C), jnp.float32), x[: L - d, :]], axis=0)
        acc = acc + sh * w[K - 1 - d:K - d, :]
    o_ref[0] = (acc * _sigmoid(acc)).astype(o_ref.dtype)


def _conv_silu(zxbcdt_3d, w_klc, conv_b, *, col0, conv_dim, cc=256):
    Bsz, L, _ = zxbcdt_3d.shape
    K = w_klc.shape[0]
    assert col0 % cc == 0 and conv_dim % cc == 0
    t0 = col0 // cc
    return pl.pallas_call(
        _conv_kernel,
        out_shape=jax.ShapeDtypeStruct((Bsz, L, conv_dim), jnp.bfloat16),
        grid=(Bsz, conv_dim // cc),
        in_specs=[
            pl.BlockSpec((1, L, cc), lambda b, c: (b, 0, t0 + c)),
            pl.BlockSpec((K, 1, cc), lambda b, c: (0, 0, c)),
            pl.BlockSpec((1, cc), lambda b, c: (0, c)),
        ],
        out_specs=pl.BlockSpec((1, L, cc), lambda b, c: (b, 0, c)),
        compiler_params=pltpu.CompilerParams(
            dimension_semantics=("parallel", "arbitrary")),
    )(zxbcdt_3d, w_klc, conv_b.reshape(1, conv_dim))


def _ssd_kernel(A_ref, D_ref, dtb_ref, dt_ref, xbc_ref, Bm_ref, Cm_ref,
                e1_ref, e2_ref, y_ref, state_ref, xw_ref, *, headdim):
    P = headdim
    HT = A_ref.shape[-1]
    Q = xbc_ref.shape[1]

    @pl.when(pl.program_id(2) == 0)
    def _init():
        state_ref[...] = jnp.zeros_like(state_ref)

    A = A_ref[0]
    Dv = D_ref[0]
    dtb = dtb_ref[0]
    dt_raw = dt_ref[0, 0]
    x = xbc_ref[0]
    Bg = Bm_ref[0]
    Cg = Cm_ref[0]
    E1 = e1_ref[...]
    E2 = e2_ref[...]

    dt = _softplus(dt_raw + dtb)
    a = dt * A

    idx_i = lax.broadcasted_iota(jnp.int32, (Q, Q), 0)
    idx_j = lax.broadcasted_iota(jnp.int32, (Q, Q), 1)
    causal = idx_i >= idx_j
    tri = causal.astype(jnp.float32)

    cA = jnp.dot(tri, a, preferred_element_type=jnp.float32)
    cAT = cA.T
    exp_cA = jnp.exp(cA)
    cA_last = cA[Q - 1:Q, :]
    exp_last = jnp.exp(cA_last)
    w_all = jnp.exp(cA_last - cA) * dt

    dtP = jnp.dot(dt, E1, preferred_element_type=jnp.float32)
    expP = jnp.dot(exp_cA, E1, preferred_element_type=jnp.float32)
    wP = jnp.dot(w_all, E1, preferred_element_type=jnp.float32)
    DvP = jnp.dot(Dv, E1, preferred_element_type=jnp.float32)
    elP = jnp.dot(exp_last, E1, preferred_element_type=jnp.float32)
    M2 = jnp.dot(cA, E2, preferred_element_type=jnp.float32)

    xf = x.astype(jnp.float32)
    xdt_bf = (xf * dtP).astype(jnp.bfloat16)
    xw_ref[...] = (xf * wP).astype(jnp.bfloat16)

    BgT = Bg.T
    scores = jnp.dot(Cg, BgT, preferred_element_type=jnp.float32)
    y_inter = jnp.dot(Cg, state_ref[...].astype(jnp.bfloat16),
                      preferred_element_type=jnp.float32)

    neg_big = jnp.float32(-1e30)
    for h in range(HT):
        sl = slice(h * P, (h + 1) * P)
        sq = slice(h * Q, (h + 1) * Q)
        diff = M2[:, sq] - cAT[h:h + 1, :]
        dec = jnp.exp(jnp.where(causal, diff, neg_big))
        y_ref[0, :, sl] = jnp.dot((scores * dec).astype(jnp.bfloat16),
                                  xdt_bf[:, sl],
                                  preferred_element_type=jnp.float32)

    y_ref[0] = y_ref[0] + expP * y_inter + DvP * xf

    dS = jnp.dot(BgT, xw_ref[...], preferred_element_type=jnp.float32)
    state_ref[...] = elP * state_ref[...] + dS


def _ssd_scan(xbc, dt_t, A, Dv, dtb, *, nheads, headdim, d_state, chunk):
    Bsz, L, _ = xbc.shape
    H, P, N, Q = nheads, headdim, d_state, chunk
    d_inner = H * P
    HT = dt_t.shape[-1]
    n_tiles = H // HT
    nC = L // Q
    bcol = d_inner // (HT * P)
    assert d_inner % (HT * P) == 0 and L % Q == 0

    hh = jnp.arange(HT, dtype=jnp.int32)[:, None]
    E1 = (jnp.arange(HT * P, dtype=jnp.int32)[None, :] // P
          == hh).astype(jnp.float32)
    E2 = (jnp.arange(HT * Q, dtype=jnp.int32)[None, :] // Q
          == hh).astype(jnp.float32)

    kfn = functools.partial(_ssd_kernel, headdim=P)
    return pl.pallas_call(
        kfn,
        out_shape=jax.ShapeDtypeStruct((Bsz, L, d_inner), jnp.float32),
        grid=(Bsz, n_tiles, nC),
        in_specs=[
            pl.BlockSpec((1, 1, HT), lambda b, t, c: (t, 0, 0)),
            pl.BlockSpec((1, 1, HT), lambda b, t, c: (t, 0, 0)),
            pl.BlockSpec((1, 1, HT), lambda b, t, c: (t, 0, 0)),
            pl.BlockSpec((1, 1, Q, HT), lambda b, t, c: (b, t, c, 0)),
            pl.BlockSpec((1, Q, HT * P), lambda b, t, c: (b, c, t)),
            pl.BlockSpec((1, Q, N),
                         lambda b, t, c: (b, c, bcol * (HT * P) // N)),
            pl.BlockSpec((1, Q, N),
                         lambda b, t, c: (b, c, bcol * (HT * P) // N + 1)),
            pl.BlockSpec((HT, HT * P), lambda b, t, c: (0, 0)),
            pl.BlockSpec((HT, HT * Q), lambda b, t, c: (0, 0)),
        ],
        out_specs=pl.BlockSpec((1, Q, HT * P), lambda b, t, c: (b, c, t)),
        scratch_shapes=[pltpu.VMEM((N, HT * P), jnp.float32),
                        pltpu.VMEM((Q, HT * P), jnp.bfloat16)],
        compiler_params=pltpu.CompilerParams(
            dimension_semantics=("parallel", "arbitrary", "arbitrary"),
            vmem_limit_bytes=24 * 1024 * 1024),
    )(A, Dv, dtb, dt_t, xbc, xbc, xbc, E1, E2)


def _norm_proj_kernel(y_ref, z_ref, nw_ref, w_ref, o_ref):
    y = y_ref[...]
    z = z_ref[...]
    x = y * (z * _sigmoid(z))
    var = jnp.mean(x * x, axis=-1, keepdims=True)
    xn = x * lax.rsqrt(var + 1e-5) * nw_ref[...]
    o_ref[...] = jnp.dot(xn.astype(jnp.bfloat16), w_ref[...],
                         preferred_element_type=jnp.float32)


def _norm_proj(y2d, z_src, norm_w, w_bf16, *, tm=128):
    M, D = y2d.shape
    _, N = w_bf16.shape
    return pl.pallas_call(
        _norm_proj_kernel,
        out_shape=jax.ShapeDtypeStruct((M, N), jnp.float32),
        grid=(M // tm,),
        in_specs=[
            pl.BlockSpec((tm, D), lambda i: (i, 0)),
            pl.BlockSpec((tm, D), lambda i: (i, 0)),
            pl.BlockSpec((1, D), lambda i: (0, 0)),
            pl.BlockSpec((D, N), lambda i: (0, 0)),
        ],
        out_specs=pl.BlockSpec((tm, N), lambda i: (i, 0)),
        compiler_params=pltpu.CompilerParams(
            dimension_semantics=("parallel",),
            vmem_limit_bytes=56 * 1024 * 1024),
    )(y2d, z_src, norm_w.reshape(1, D), w_bf16)


def kernel(u, in_proj_wT, conv_w_klc, conv_b, A_log, D, dt_bias, norm_w,
           out_proj_wT):
    d_model, d_inner, d_state = 2048, 4096, 128
    H, P, G, K = 64, 64, 1, 4
    HT = 16
    chunk = 128
    n_tiles = H // HT
    conv_dim = d_inner + 2 * G * d_state
    d_in_proj = 2 * d_inner + 2 * G * d_state + H

    Bsz, L, _ = u.shape
    M = Bsz * L

    zxbcdt = _inproj(u.reshape(M, d_model).astype(jnp.bfloat16), in_proj_wT)

    xbc = _conv_silu(zxbcdt.reshape(Bsz, L, d_in_proj), conv_w_klc, conv_b,
                     col0=d_inner, conv_dim=conv_dim)

    dt_raw = zxbcdt[:, d_inner + conv_dim:]
    dt_t = dt_raw.reshape(Bsz, L, n_tiles, HT).transpose(0, 2, 1, 3)

    A = (-jnp.exp(A_log)).reshape(n_tiles, 1, HT).astype(jnp.float32)
    Dv = D.reshape(n_tiles, 1, HT).astype(jnp.float32)
    dtb = dt_bias.reshape(n_tiles, 1, HT).astype(jnp.float32)

    y = _ssd_scan(xbc, dt_t, A, Dv, dtb, nheads=H, headdim=P,
                  d_state=d_state, chunk=chunk)

    out = _norm_proj(y.reshape(M, d_inner), zxbcdt, norm_w, out_proj_wT)
    return out.reshape(Bsz, L, d_model)
